# baseline (speedup 1.0000x reference)
import functools

import jax
import jax.numpy as jnp
from jax import lax
from jax.experimental import pallas as pl
from jax.experimental.pallas import tpu as pltpu, tpu_sc as plsc

GATHER = 128
NBUF = 5


@functools.lru_cache(maxsize=None)
def _build(B: int, D: int):
    info = plsc.get_sparse_core_info()
    NC, NS = info.num_cores, info.num_subcores
    NW = NC * NS
    assert B % (NW * GATHER * NBUF) == 0
    b_per_w = B // NW
    n_steps = b_per_w // GATHER
    n_groups = n_steps // NBUF

    mesh = plsc.VectorSubcoreMesh(core_axis_name="c", subcore_axis_name="s")

    @functools.partial(
        pl.kernel,
        out_type=jax.ShapeDtypeStruct((B, D), jnp.float32),
        mesh=mesh,
        scratch_types=[
            pltpu.VMEM((b_per_w,), jnp.int32),
            pltpu.VMEM((NBUF, GATHER, 2 * D), jnp.float32),
        ] + [pltpu.SemaphoreType.DMA] * (2 * NBUF),
        compiler_params=pltpu.CompilerParams(
            use_tc_tiling_on_sc=False, needs_layout_passes=False
        ),
    )
    def emb(table_hbm, idx_hbm, out_hbm, idx_v, rows_v, *sems):
        gsem, wsem = sems[:NBUF], sems[NBUF:]
        wid = lax.axis_index("s") * NC + lax.axis_index("c")
        base = wid * b_per_w
        pltpu.sync_copy(idx_hbm.at[pl.ds(base, b_per_w)], idx_v)

        def gather_desc(step, b, sem):
            return pltpu.make_async_copy(
                table_hbm.at[idx_v.at[pl.ds(step * GATHER, GATHER)]],
                rows_v.at[b],
                sem,
            )

        def write_desc(step, b, sem):
            return pltpu.make_async_copy(
                rows_v.at[b, :, pl.ds(0, D)],
                out_hbm.at[pl.ds(base + step * GATHER, GATHER)],
                sem,
            )

        for b in range(NBUF):
            gather_desc(b, b, gsem[b]).start()

        @pl.loop(0, n_groups - 1)
        def grp(k):
            for b in range(NBUF):
                i = k * NBUF + b
                gather_desc(i, b, gsem[b]).wait()
                write_desc(i, b, wsem[b]).start()
                write_desc(i, b, wsem[b]).wait()
                gather_desc(i + NBUF, b, gsem[b]).start()

        for b in range(NBUF):
            i = (n_groups - 1) * NBUF + b
            gather_desc(i, b, gsem[b]).wait()
            write_desc(i, b, wsem[b]).start()
        for b in range(NBUF):
            i = (n_groups - 1) * NBUF + b
            write_desc(i, b, wsem[b]).wait()

    return emb


def kernel(x, table):
    A, Bdim = x.shape
    D = table.shape[1]
    xf = jnp.swapaxes(x, 0, 1).reshape(-1).astype(jnp.int32)
    tbl = jnp.pad(table, ((0, 0), (0, D)))
    out = _build(xf.shape[0], D)(tbl, xf)
    return out.reshape(Bdim, A, D).transpose(1, 0, 2)


# speedup vs baseline: 1.0098x; 1.0098x over previous
import functools

import jax
import jax.numpy as jnp
from jax import lax
from jax.experimental import pallas as pl
from jax.experimental.pallas import tpu as pltpu, tpu_sc as plsc

GATHER = 256
NBUF = 5


@functools.lru_cache(maxsize=None)
def _build(B: int, D: int):
    info = plsc.get_sparse_core_info()
    NC, NS = info.num_cores, info.num_subcores
    NW = NC * NS
    assert B % (NW * GATHER * NBUF) == 0
    b_per_w = B // NW
    n_steps = b_per_w // GATHER
    n_groups = n_steps // NBUF

    mesh = plsc.VectorSubcoreMesh(core_axis_name="c", subcore_axis_name="s")

    @functools.partial(
        pl.kernel,
        out_type=jax.ShapeDtypeStruct((B, D), jnp.float32),
        mesh=mesh,
        scratch_types=[
            pltpu.VMEM((b_per_w,), jnp.int32),
            pltpu.VMEM((NBUF, GATHER, D), jnp.float32),
        ] + [pltpu.SemaphoreType.DMA] * (2 * NBUF),
        compiler_params=pltpu.CompilerParams(
            use_tc_tiling_on_sc=False, needs_layout_passes=False
        ),
    )
    def emb(table_hbm, idx_hbm, out_hbm, idx_v, rows_v, *sems):
        gsem, wsem = sems[:NBUF], sems[NBUF:]
        wid = lax.axis_index("s") * NC + lax.axis_index("c")
        base = wid * b_per_w
        pltpu.sync_copy(idx_hbm.at[pl.ds(base, b_per_w)], idx_v)

        def gather_desc(step, b, sem):
            return pltpu.make_async_copy(
                table_hbm.at[idx_v.at[pl.ds(step * GATHER, GATHER)]],
                rows_v.at[b],
                sem,
            )

        def write_desc(step, b, sem):
            return pltpu.make_async_copy(
                rows_v.at[b],
                out_hbm.at[pl.ds(base + step * GATHER, GATHER)],
                sem,
            )

        for b in range(NBUF):
            gather_desc(b, b, gsem[b]).start()

        @pl.loop(0, n_groups - 1)
        def grp(k):
            for b in range(NBUF):
                i = k * NBUF + b
                gather_desc(i, b, gsem[b]).wait()
                write_desc(i, b, wsem[b]).start()
                write_desc(i, b, wsem[b]).wait()
                gather_desc(i + NBUF, b, gsem[b]).start()

        for b in range(NBUF):
            i = (n_groups - 1) * NBUF + b
            gather_desc(i, b, gsem[b]).wait()
            write_desc(i, b, wsem[b]).start()
        for b in range(NBUF):
            i = (n_groups - 1) * NBUF + b
            write_desc(i, b, wsem[b]).wait()

    return emb


def kernel(x, table):
    A, Bdim = x.shape
    D = table.shape[1]
    xf = jnp.swapaxes(x, 0, 1).reshape(-1).astype(jnp.int32)
    tbl_lin = lax.optimization_barrier(
        table.reshape(table.shape[0] // 2, 2 * D)
    )
    tbl = tbl_lin.reshape(table.shape)
    out = _build(xf.shape[0], D)(tbl, xf)
    return out.reshape(Bdim, A, D).transpose(1, 0, 2)
